# Initial kernel scaffold; baseline (speedup 1.0000x reference)
import jax
import jax.numpy as jnp
from jax.experimental import pallas as pl


def kernel(boxes, scores):
    raise NotImplementedError("write your pallas kernel here")


# speedup vs baseline: 22.1640x; 22.1640x over previous
import jax
import jax.numpy as jnp
from jax import lax
from jax.experimental import pallas as pl
from jax.experimental.pallas import tpu as pltpu
from jax.experimental.pallas import tpu_sc as plsc

N = 5000
N_PAD = 5120
NUM_TECS = 16
NB = N_PAD // 16
ROWS_PER_TEC = N_PAD // NUM_TECS
SCORE_THRESHOLD = 0.05
NMS_THRESHOLD = 0.5

_f32 = jnp.float32
_i32 = jnp.int32


def _nms_body(x1h, y1h, x2h, y2h, ph,
              o_x1, o_y1, o_x2, o_y2, o_p,
              vx1, vy1, vx2, vy2, vp,
              lx1, ly1, lx2, ly2, la,
              kept_all, tmp16, pall, kf_v,
              vo0, vo1, vo2, vo3, vo4,
              sh_part, sh_kf):
    c = lax.axis_index("c")
    w = lax.axis_index("s")

    def bcast(v, t):
        return jnp.full((16,), v[t], v.dtype)

    @pl.when(c == 0)
    def _():
        pltpu.sync_copy(x1h, vx1)
        pltpu.sync_copy(y1h, vy1)
        pltpu.sync_copy(x2h, vx2)
        pltpu.sync_copy(y2h, vy2)
        pltpu.sync_copy(ph, vp)

        zf = jnp.zeros((16,), _f32)
        for g in range((N_PAD + 16) // 16):
            for ref in (lx1, ly1, lx2, ly2, la):
                ref[pl.ds(16 * g, 16)] = zf

        lanes = lax.iota(_i32, 16)

        def block_body(b, l_len):
            base = 16 * b
            cx1 = vx1[pl.ds(base, 16)]
            cy1 = vy1[pl.ds(base, 16)]
            cx2 = vx2[pl.ds(base, 16)]
            cy2 = vy2[pl.ds(base, 16)]
            cp = vp[pl.ds(base, 16)]
            carea = jnp.maximum(cx2 - cx1, 0.0) * jnp.maximum(cy2 - cy1, 0.0)

            nchunks = (l_len + 15) // 16
            nj = jnp.maximum(nchunks - w + 15, 0) // 16

            def chunk_body(j, sup):
                r0 = 16 * (w + 16 * j)
                rx1v = lx1[pl.ds(r0, 16)]
                ry1v = ly1[pl.ds(r0, 16)]
                rx2v = lx2[pl.ds(r0, 16)]
                ry2v = ly2[pl.ds(r0, 16)]
                rav = la[pl.ds(r0, 16)]
                for t in range(16):
                    xx1 = jnp.maximum(cx1, bcast(rx1v, t))
                    yy1 = jnp.maximum(cy1, bcast(ry1v, t))
                    xx2 = jnp.minimum(cx2, bcast(rx2v, t))
                    yy2 = jnp.minimum(cy2, bcast(ry2v, t))
                    inter = (jnp.maximum(xx2 - xx1, 0.0)
                             * jnp.maximum(yy2 - yy1, 0.0))
                    union = carea + bcast(rav, t) - inter
                    sup = sup | jnp.where(inter > NMS_THRESHOLD * union,
                                          1, 0)
                return sup

            sup = lax.fori_loop(0, nj, chunk_body,
                                jnp.zeros((16,), _i32))

            tmp16[...] = sup
            pltpu.sync_copy(tmp16, sh_part.at[pl.ds(16 * w, 16)])
            plsc.subcore_barrier()

            @pl.when(w == 0)
            def _():
                pltpu.sync_copy(sh_part, pall)
                acc = pall[pl.ds(0, 16)]
                for i in range(1, NUM_TECS):
                    acc = acc | pall[pl.ds(16 * i, 16)]
                kept = jnp.where((cp >= SCORE_THRESHOLD) & (acc == 0),
                                 1, 0)
                for t in range(16):
                    xx1 = jnp.maximum(cx1, bcast(cx1, t))
                    yy1 = jnp.maximum(cy1, bcast(cy1, t))
                    xx2 = jnp.minimum(cx2, bcast(cx2, t))
                    yy2 = jnp.minimum(cy2, bcast(cy2, t))
                    inter = (jnp.maximum(xx2 - xx1, 0.0)
                             * jnp.maximum(yy2 - yy1, 0.0))
                    union = carea + bcast(carea, t) - inter
                    ktv = jnp.full((16,), kept[t], _i32)
                    m = (inter > NMS_THRESHOLD * union) & (lanes > t)
                    supt = jnp.where(m, ktv, 0)
                    kept = kept * (1 - supt)
                tmp16[...] = kept
                pltpu.sync_copy(tmp16, sh_kf)

            plsc.subcore_barrier()

            pltpu.sync_copy(sh_kf, kf_v)
            keptv = kf_v[...]
            kept_all[pl.ds(base, 16)] = keptv
            pos = keptv[0] * 0
            ax1 = jnp.zeros((16,), _f32)
            ay1 = jnp.zeros((16,), _f32)
            ax2 = jnp.zeros((16,), _f32)
            ay2 = jnp.zeros((16,), _f32)
            aa = jnp.zeros((16,), _f32)
            for t in range(16):
                kt = keptv[t]
                ktf = kt.astype(_f32)
                hit = lanes == jnp.full((16,), pos, _i32)
                mf = jnp.where(hit, ktf, 0.0)
                ax1 = ax1 + mf * (bcast(cx1, t) - ax1)
                ay1 = ay1 + mf * (bcast(cy1, t) - ay1)
                ax2 = ax2 + mf * (bcast(cx2, t) - ax2)
                ay2 = ay2 + mf * (bcast(cy2, t) - ay2)
                aa = aa + mf * (bcast(carea, t) - aa)
                pos = pos + kt
            lx1[pl.ds(l_len, 16)] = ax1
            ly1[pl.ds(l_len, 16)] = ay1
            lx2[pl.ds(l_len, 16)] = ax2
            ly2[pl.ds(l_len, 16)] = ay2
            la[pl.ds(l_len, 16)] = aa
            return l_len + pos

        lax.fori_loop(0, NB, block_body, jnp.int32(0))

        row0 = ROWS_PER_TEC * w
        for g in range(ROWS_PER_TEC // 16):
            idx = row0 + 16 * g
            keepf = jnp.where(kept_all[pl.ds(idx, 16)] != 0, 1.0, 0.0)
            vo0[pl.ds(16 * g, 16)] = vx1[pl.ds(idx, 16)] * keepf
            vo1[pl.ds(16 * g, 16)] = vy1[pl.ds(idx, 16)] * keepf
            vo2[pl.ds(16 * g, 16)] = vx2[pl.ds(idx, 16)] * keepf
            vo3[pl.ds(16 * g, 16)] = vy2[pl.ds(idx, 16)] * keepf
            vo4[pl.ds(16 * g, 16)] = vp[pl.ds(idx, 16)] * keepf
        pltpu.sync_copy(vo0, o_x1.at[pl.ds(row0, ROWS_PER_TEC)])
        pltpu.sync_copy(vo1, o_y1.at[pl.ds(row0, ROWS_PER_TEC)])
        pltpu.sync_copy(vo2, o_x2.at[pl.ds(row0, ROWS_PER_TEC)])
        pltpu.sync_copy(vo3, o_y2.at[pl.ds(row0, ROWS_PER_TEC)])
        pltpu.sync_copy(vo4, o_p.at[pl.ds(row0, ROWS_PER_TEC)])


@jax.jit
def kernel(boxes, scores):
    probs = jax.nn.sigmoid(scores)
    order = jnp.argsort(-probs)
    b = jnp.take(boxes, order, axis=0)
    p = jnp.take(probs, order, axis=0)

    pad = N_PAD - N
    x1 = jnp.pad(b[:, 0], (0, pad))
    y1 = jnp.pad(b[:, 1], (0, pad))
    x2 = jnp.pad(b[:, 2], (0, pad))
    y2 = jnp.pad(b[:, 3], (0, pad))
    pp = jnp.pad(p, (0, pad))

    mesh = plsc.VectorSubcoreMesh(core_axis_name="c", subcore_axis_name="s")
    f = pl.kernel(
        _nms_body,
        out_type=[jax.ShapeDtypeStruct((N_PAD,), _f32)] * 5,
        mesh=mesh,
        scratch_types=[
            pltpu.VMEM((N_PAD,), _f32),
            pltpu.VMEM((N_PAD,), _f32),
            pltpu.VMEM((N_PAD,), _f32),
            pltpu.VMEM((N_PAD,), _f32),
            pltpu.VMEM((N_PAD,), _f32),
            pltpu.VMEM((N_PAD + 16,), _f32),
            pltpu.VMEM((N_PAD + 16,), _f32),
            pltpu.VMEM((N_PAD + 16,), _f32),
            pltpu.VMEM((N_PAD + 16,), _f32),
            pltpu.VMEM((N_PAD + 16,), _f32),
            pltpu.VMEM((N_PAD,), _i32),
            pltpu.VMEM((16,), _i32),
            pltpu.VMEM((NUM_TECS * 16,), _i32),
            pltpu.VMEM((16,), _i32),
            pltpu.VMEM((ROWS_PER_TEC,), _f32),
            pltpu.VMEM((ROWS_PER_TEC,), _f32),
            pltpu.VMEM((ROWS_PER_TEC,), _f32),
            pltpu.VMEM((ROWS_PER_TEC,), _f32),
            pltpu.VMEM((ROWS_PER_TEC,), _f32),
            pltpu.VMEM_SHARED((NUM_TECS * 16,), _i32),
            pltpu.VMEM_SHARED((16,), _i32),
        ],
    )
    o_x1, o_y1, o_x2, o_y2, o_p = f(x1, y1, x2, y2, pp)
    out = jnp.stack([o_x1, o_y1, o_x2, o_y2, o_p], axis=1)
    return out[:N]


# speedup vs baseline: 22.5068x; 1.0155x over previous
import jax
import jax.numpy as jnp
from jax import lax
from jax.experimental import pallas as pl
from jax.experimental.pallas import tpu as pltpu
from jax.experimental.pallas import tpu_sc as plsc

N = 5000
N_PAD = 5120
NUM_TECS = 16
NB = N_PAD // 16
ROWS_PER_TEC = N_PAD // NUM_TECS
SCORE_THRESHOLD = 0.05
NMS_THRESHOLD = 0.5

_f32 = jnp.float32
_i32 = jnp.int32


def _nms_body(x1h, y1h, x2h, y2h, ph,
              o_x1, o_y1, o_x2, o_y2, o_p,
              vx1, vy1, vx2, vy2, vp,
              lx1, ly1, lx2, ly2, la,
              kept_all, tmp32, pall, pub_v,
              vo0, vo1, vo2, vo3, vo4,
              sh_part, sh_pub):
    c = lax.axis_index("c")
    w = lax.axis_index("s")

    def bcast(v, t):
        return jnp.full((16,), v[t], v.dtype)

    @pl.when(c == 0)
    def _():
        pltpu.sync_copy(x1h, vx1.at[pl.ds(0, N_PAD)])
        pltpu.sync_copy(y1h, vy1.at[pl.ds(0, N_PAD)])
        pltpu.sync_copy(x2h, vx2.at[pl.ds(0, N_PAD)])
        pltpu.sync_copy(y2h, vy2.at[pl.ds(0, N_PAD)])
        pltpu.sync_copy(ph, vp)

        zf = jnp.zeros((16,), _f32)
        for g in range((N_PAD + 16) // 16):
            for ref in (lx1, ly1, lx2, ly2, la):
                ref[pl.ds(16 * g, 16)] = zf

        lanes = lax.iota(_i32, 16)

        def block_body(b, l_len):
            base = 16 * b
            cx1 = vx1[pl.ds(base, 16)]
            cy1 = vy1[pl.ds(base, 16)]
            cx2 = vx2[pl.ds(base, 16)]
            cy2 = vy2[pl.ds(base, 16)]
            cp = vp[pl.ds(base, 16)]
            carea = jnp.maximum(cx2 - cx1, 0.0) * jnp.maximum(cy2 - cy1, 0.0)

            nchunks = (l_len + 15) // 16
            nj = jnp.maximum(nchunks - w + 15, 0) // 16

            def chunk_body(j, sup):
                r0 = 16 * (w + 16 * j)
                rx1v = lx1[pl.ds(r0, 16)]
                ry1v = ly1[pl.ds(r0, 16)]
                rx2v = lx2[pl.ds(r0, 16)]
                ry2v = ly2[pl.ds(r0, 16)]
                rav = la[pl.ds(r0, 16)]
                for t in range(16):
                    xx1 = jnp.maximum(cx1, bcast(rx1v, t))
                    yy1 = jnp.maximum(cy1, bcast(ry1v, t))
                    xx2 = jnp.minimum(cx2, bcast(rx2v, t))
                    yy2 = jnp.minimum(cy2, bcast(ry2v, t))
                    inter = (jnp.maximum(xx2 - xx1, 0.0)
                             * jnp.maximum(yy2 - yy1, 0.0))
                    union = carea + bcast(rav, t) - inter
                    sup = sup | jnp.where(inter > NMS_THRESHOLD * union,
                                          1, 0)
                return sup

            sup = lax.fori_loop(0, nj, chunk_body,
                                jnp.zeros((16,), _i32))

            wrow = vx1[pl.ds(base + w, 16)]
            xw1 = jnp.full((16,), wrow[0], _f32)
            yw1 = jnp.full((16,), vy1[pl.ds(base + w, 16)][0], _f32)
            xw2 = jnp.full((16,), vx2[pl.ds(base + w, 16)][0], _f32)
            yw2 = jnp.full((16,), vy2[pl.ds(base + w, 16)][0], _f32)
            aw = (jnp.maximum(xw2 - xw1, 0.0)
                  * jnp.maximum(yw2 - yw1, 0.0))
            xx1 = jnp.maximum(cx1, xw1)
            yy1 = jnp.maximum(cy1, yw1)
            xx2 = jnp.minimum(cx2, xw2)
            yy2 = jnp.minimum(cy2, yw2)
            inter = (jnp.maximum(xx2 - xx1, 0.0)
                     * jnp.maximum(yy2 - yy1, 0.0))
            union = carea + aw - inter
            wv = jnp.full((16,), w, _i32)
            rowm = (jnp.where(inter > NMS_THRESHOLD * union, 1, 0)
                    * jnp.where(lanes > wv, 1, 0))

            tmp32[pl.ds(0, 16)] = sup
            tmp32[pl.ds(16, 16)] = rowm
            pltpu.sync_copy(tmp32, sh_part.at[pl.ds(32 * w, 32)])
            plsc.subcore_barrier()

            @pl.when(w == 0)
            def _():
                pltpu.sync_copy(sh_part, pall)
                acc = pall[pl.ds(0, 16)]
                for i in range(1, NUM_TECS):
                    acc = acc | pall[pl.ds(32 * i, 16)]
                kept = jnp.where((cp >= SCORE_THRESHOLD) & (acc == 0),
                                 1, 0)
                for t in range(16):
                    rowt = pall[pl.ds(32 * t + 16, 16)]
                    ktv = jnp.full((16,), kept[t], _i32)
                    kept = kept * (1 - rowt * ktv)
                pos = kept[0] * 0
                ax1 = jnp.zeros((16,), _f32)
                ay1 = jnp.zeros((16,), _f32)
                ax2 = jnp.zeros((16,), _f32)
                ay2 = jnp.zeros((16,), _f32)
                aa = jnp.zeros((16,), _f32)
                for t in range(16):
                    kt = kept[t]
                    ktf = kt.astype(_f32)
                    hit = lanes == jnp.full((16,), pos, _i32)
                    mf = jnp.where(hit, ktf, 0.0)
                    ax1 = ax1 + mf * (bcast(cx1, t) - ax1)
                    ay1 = ay1 + mf * (bcast(cy1, t) - ay1)
                    ax2 = ax2 + mf * (bcast(cx2, t) - ax2)
                    ay2 = ay2 + mf * (bcast(cy2, t) - ay2)
                    aa = aa + mf * (bcast(carea, t) - aa)
                    pos = pos + kt
                pub_v[pl.ds(0, 16)] = jnp.where(kept != 0, 1.0, 0.0)
                pub_v[pl.ds(16, 16)] = jnp.full((16,), pos.astype(_f32),
                                                _f32)
                pub_v[pl.ds(32, 16)] = ax1
                pub_v[pl.ds(48, 16)] = ay1
                pub_v[pl.ds(64, 16)] = ax2
                pub_v[pl.ds(80, 16)] = ay2
                pub_v[pl.ds(96, 16)] = aa
                pltpu.sync_copy(pub_v, sh_pub)

            plsc.subcore_barrier()

            pltpu.sync_copy(sh_pub, pub_v)
            keptv = jnp.where(pub_v[pl.ds(0, 16)] != 0.0, 1, 0)
            kept_all[pl.ds(base, 16)] = keptv
            cnt = pub_v[pl.ds(16, 16)][0].astype(_i32)
            lx1[pl.ds(l_len, 16)] = pub_v[pl.ds(32, 16)]
            ly1[pl.ds(l_len, 16)] = pub_v[pl.ds(48, 16)]
            lx2[pl.ds(l_len, 16)] = pub_v[pl.ds(64, 16)]
            ly2[pl.ds(l_len, 16)] = pub_v[pl.ds(80, 16)]
            la[pl.ds(l_len, 16)] = pub_v[pl.ds(96, 16)]
            return l_len + cnt

        lax.fori_loop(0, NB, block_body, jnp.int32(0))

        row0 = ROWS_PER_TEC * w
        for g in range(ROWS_PER_TEC // 16):
            idx = row0 + 16 * g
            keepf = jnp.where(kept_all[pl.ds(idx, 16)] != 0, 1.0, 0.0)
            vo0[pl.ds(16 * g, 16)] = vx1[pl.ds(idx, 16)] * keepf
            vo1[pl.ds(16 * g, 16)] = vy1[pl.ds(idx, 16)] * keepf
            vo2[pl.ds(16 * g, 16)] = vx2[pl.ds(idx, 16)] * keepf
            vo3[pl.ds(16 * g, 16)] = vy2[pl.ds(idx, 16)] * keepf
            vo4[pl.ds(16 * g, 16)] = vp[pl.ds(idx, 16)] * keepf
        pltpu.sync_copy(vo0, o_x1.at[pl.ds(row0, ROWS_PER_TEC)])
        pltpu.sync_copy(vo1, o_y1.at[pl.ds(row0, ROWS_PER_TEC)])
        pltpu.sync_copy(vo2, o_x2.at[pl.ds(row0, ROWS_PER_TEC)])
        pltpu.sync_copy(vo3, o_y2.at[pl.ds(row0, ROWS_PER_TEC)])
        pltpu.sync_copy(vo4, o_p.at[pl.ds(row0, ROWS_PER_TEC)])


@jax.jit
def kernel(boxes, scores):
    probs = jax.nn.sigmoid(scores)
    order = jnp.argsort(-probs)
    b = jnp.take(boxes, order, axis=0)
    p = jnp.take(probs, order, axis=0)

    pad = N_PAD - N
    x1 = jnp.pad(b[:, 0], (0, pad))
    y1 = jnp.pad(b[:, 1], (0, pad))
    x2 = jnp.pad(b[:, 2], (0, pad))
    y2 = jnp.pad(b[:, 3], (0, pad))
    pp = jnp.pad(p, (0, pad))

    mesh = plsc.VectorSubcoreMesh(core_axis_name="c", subcore_axis_name="s")
    f = pl.kernel(
        _nms_body,
        out_type=[jax.ShapeDtypeStruct((N_PAD,), _f32)] * 5,
        mesh=mesh,
        scratch_types=[
            pltpu.VMEM((N_PAD + 16,), _f32),
            pltpu.VMEM((N_PAD + 16,), _f32),
            pltpu.VMEM((N_PAD + 16,), _f32),
            pltpu.VMEM((N_PAD + 16,), _f32),
            pltpu.VMEM((N_PAD,), _f32),
            pltpu.VMEM((N_PAD + 16,), _f32),
            pltpu.VMEM((N_PAD + 16,), _f32),
            pltpu.VMEM((N_PAD + 16,), _f32),
            pltpu.VMEM((N_PAD + 16,), _f32),
            pltpu.VMEM((N_PAD + 16,), _f32),
            pltpu.VMEM((N_PAD,), _i32),
            pltpu.VMEM((32,), _i32),
            pltpu.VMEM((NUM_TECS * 32,), _i32),
            pltpu.VMEM((112,), _f32),
            pltpu.VMEM((ROWS_PER_TEC,), _f32),
            pltpu.VMEM((ROWS_PER_TEC,), _f32),
            pltpu.VMEM((ROWS_PER_TEC,), _f32),
            pltpu.VMEM((ROWS_PER_TEC,), _f32),
            pltpu.VMEM((ROWS_PER_TEC,), _f32),
            pltpu.VMEM_SHARED((NUM_TECS * 32,), _i32),
            pltpu.VMEM_SHARED((112,), _f32),
        ],
    )
    o_x1, o_y1, o_x2, o_y2, o_p = f(x1, y1, x2, y2, pp)
    out = jnp.stack([o_x1, o_y1, o_x2, o_y2, o_p], axis=1)
    return out[:N]


# speedup vs baseline: 32.3825x; 1.4388x over previous
import jax
import jax.numpy as jnp
from jax import lax
from jax.experimental import pallas as pl
from jax.experimental.pallas import tpu as pltpu
from jax.experimental.pallas import tpu_sc as plsc

N = 5000
N_PAD = 5120
NUM_TECS = 16
NB = N_PAD // 16
ROWS_PER_TEC = N_PAD // NUM_TECS
SCORE_THRESHOLD = 0.05
NMS_THRESHOLD = 0.5

_f32 = jnp.float32
_i32 = jnp.int32


def _nms_body(x1h, y1h, x2h, y2h, ph,
              o_x1, o_y1, o_x2, o_y2, o_p,
              vx1, vy1, vx2, vy2, vp,
              lx1, ly1, lx2, ly2, la,
              kept_all, tmp32, pall, pub_v,
              vo0, vo1, vo2, vo3, vo4,
              sh_part, sh_pub):
    c = lax.axis_index("c")
    w = lax.axis_index("s")

    def bcast(v, t):
        return jnp.full((16,), v[t], v.dtype)

    @pl.when(c == 0)
    def _():
        pltpu.sync_copy(x1h, vx1.at[pl.ds(0, N_PAD)])
        pltpu.sync_copy(y1h, vy1.at[pl.ds(0, N_PAD)])
        pltpu.sync_copy(x2h, vx2.at[pl.ds(0, N_PAD)])
        pltpu.sync_copy(y2h, vy2.at[pl.ds(0, N_PAD)])
        pltpu.sync_copy(ph, vp)

        zf = jnp.zeros((16,), _f32)
        for g in range((N_PAD + 16) // 16):
            for ref in (lx1, ly1, lx2, ly2, la):
                ref[pl.ds(16 * g, 16)] = zf

        lanes = lax.iota(_i32, 16)

        def block_body(b, l_len):
            base = 16 * b
            cx1 = vx1[pl.ds(base, 16)]
            cy1 = vy1[pl.ds(base, 16)]
            cx2 = vx2[pl.ds(base, 16)]
            cy2 = vy2[pl.ds(base, 16)]
            cp = vp[pl.ds(base, 16)]
            carea = jnp.maximum(cx2 - cx1, 0.0) * jnp.maximum(cy2 - cy1, 0.0)

            nchunks = (l_len + 15) // 16
            nj = jnp.maximum(nchunks - w + 15, 0) // 16

            def chunk_body(j, sup):
                r0 = 16 * (w + 16 * j)
                rx1v = lx1[pl.ds(r0, 16)]
                ry1v = ly1[pl.ds(r0, 16)]
                rx2v = lx2[pl.ds(r0, 16)]
                ry2v = ly2[pl.ds(r0, 16)]
                rav = la[pl.ds(r0, 16)]
                for t in range(16):
                    xx1 = jnp.maximum(cx1, bcast(rx1v, t))
                    yy1 = jnp.maximum(cy1, bcast(ry1v, t))
                    xx2 = jnp.minimum(cx2, bcast(rx2v, t))
                    yy2 = jnp.minimum(cy2, bcast(ry2v, t))
                    inter = (jnp.maximum(xx2 - xx1, 0.0)
                             * jnp.maximum(yy2 - yy1, 0.0))
                    union = carea + bcast(rav, t) - inter
                    sup = sup | jnp.where(inter > NMS_THRESHOLD * union,
                                          1, 0)
                return sup

            sup = lax.fori_loop(0, nj, chunk_body,
                                jnp.zeros((16,), _i32))

            tmp32[pl.ds(0, 16)] = sup
            pltpu.sync_copy(tmp32.at[pl.ds(0, 16)],
                            sh_part.at[pl.ds(16 * w, 16)])
            plsc.subcore_barrier()

            pltpu.sync_copy(sh_part, pall)
            acc = pall[pl.ds(0, 16)]
            for i in range(1, NUM_TECS):
                acc = acc | pall[pl.ds(16 * i, 16)]
            kept = jnp.where((cp >= SCORE_THRESHOLD) & (acc == 0), 1, 0)
            pos = kept[0] * 0
            ax1 = jnp.zeros((16,), _f32)
            ay1 = jnp.zeros((16,), _f32)
            ax2 = jnp.zeros((16,), _f32)
            ay2 = jnp.zeros((16,), _f32)
            aa = jnp.zeros((16,), _f32)
            for t in range(16):
                xt1 = bcast(cx1, t)
                yt1 = bcast(cy1, t)
                xt2 = bcast(cx2, t)
                yt2 = bcast(cy2, t)
                at = bcast(carea, t)
                xx1 = jnp.maximum(cx1, xt1)
                yy1 = jnp.maximum(cy1, yt1)
                xx2 = jnp.minimum(cx2, xt2)
                yy2 = jnp.minimum(cy2, yt2)
                inter = (jnp.maximum(xx2 - xx1, 0.0)
                         * jnp.maximum(yy2 - yy1, 0.0))
                union = carea + at - inter
                rowt = (jnp.where(inter > NMS_THRESHOLD * union, 1, 0)
                        * jnp.where(lanes > t, 1, 0))
                kt = kept[t]
                ktv = jnp.full((16,), kt, _i32)
                kept = kept * (1 - rowt * ktv)
                ktf = kt.astype(_f32)
                hit = lanes == jnp.full((16,), pos, _i32)
                mf = jnp.where(hit, ktf, 0.0)
                ax1 = ax1 + mf * (xt1 - ax1)
                ay1 = ay1 + mf * (yt1 - ay1)
                ax2 = ax2 + mf * (xt2 - ax2)
                ay2 = ay2 + mf * (yt2 - ay2)
                aa = aa + mf * (at - aa)
                pos = pos + kt
            kept_all[pl.ds(base, 16)] = kept
            lx1[pl.ds(l_len, 16)] = ax1
            ly1[pl.ds(l_len, 16)] = ay1
            lx2[pl.ds(l_len, 16)] = ax2
            ly2[pl.ds(l_len, 16)] = ay2
            la[pl.ds(l_len, 16)] = aa
            return l_len + pos

        lax.fori_loop(0, NB, block_body, jnp.int32(0))

        row0 = ROWS_PER_TEC * w
        for g in range(ROWS_PER_TEC // 16):
            idx = row0 + 16 * g
            keepf = jnp.where(kept_all[pl.ds(idx, 16)] != 0, 1.0, 0.0)
            vo0[pl.ds(16 * g, 16)] = vx1[pl.ds(idx, 16)] * keepf
            vo1[pl.ds(16 * g, 16)] = vy1[pl.ds(idx, 16)] * keepf
            vo2[pl.ds(16 * g, 16)] = vx2[pl.ds(idx, 16)] * keepf
            vo3[pl.ds(16 * g, 16)] = vy2[pl.ds(idx, 16)] * keepf
            vo4[pl.ds(16 * g, 16)] = vp[pl.ds(idx, 16)] * keepf
        pltpu.sync_copy(vo0, o_x1.at[pl.ds(row0, ROWS_PER_TEC)])
        pltpu.sync_copy(vo1, o_y1.at[pl.ds(row0, ROWS_PER_TEC)])
        pltpu.sync_copy(vo2, o_x2.at[pl.ds(row0, ROWS_PER_TEC)])
        pltpu.sync_copy(vo3, o_y2.at[pl.ds(row0, ROWS_PER_TEC)])
        pltpu.sync_copy(vo4, o_p.at[pl.ds(row0, ROWS_PER_TEC)])


@jax.jit
def kernel(boxes, scores):
    probs = jax.nn.sigmoid(scores)
    order = jnp.argsort(-probs)
    b = jnp.take(boxes, order, axis=0)
    p = jnp.take(probs, order, axis=0)

    pad = N_PAD - N
    x1 = jnp.pad(b[:, 0], (0, pad))
    y1 = jnp.pad(b[:, 1], (0, pad))
    x2 = jnp.pad(b[:, 2], (0, pad))
    y2 = jnp.pad(b[:, 3], (0, pad))
    pp = jnp.pad(p, (0, pad))

    mesh = plsc.VectorSubcoreMesh(core_axis_name="c", subcore_axis_name="s")
    f = pl.kernel(
        _nms_body,
        out_type=[jax.ShapeDtypeStruct((N_PAD,), _f32)] * 5,
        mesh=mesh,
        scratch_types=[
            pltpu.VMEM((N_PAD + 16,), _f32),
            pltpu.VMEM((N_PAD + 16,), _f32),
            pltpu.VMEM((N_PAD + 16,), _f32),
            pltpu.VMEM((N_PAD + 16,), _f32),
            pltpu.VMEM((N_PAD,), _f32),
            pltpu.VMEM((N_PAD + 16,), _f32),
            pltpu.VMEM((N_PAD + 16,), _f32),
            pltpu.VMEM((N_PAD + 16,), _f32),
            pltpu.VMEM((N_PAD + 16,), _f32),
            pltpu.VMEM((N_PAD + 16,), _f32),
            pltpu.VMEM((N_PAD,), _i32),
            pltpu.VMEM((32,), _i32),
            pltpu.VMEM((NUM_TECS * 16,), _i32),
            pltpu.VMEM((112,), _f32),
            pltpu.VMEM((ROWS_PER_TEC,), _f32),
            pltpu.VMEM((ROWS_PER_TEC,), _f32),
            pltpu.VMEM((ROWS_PER_TEC,), _f32),
            pltpu.VMEM((ROWS_PER_TEC,), _f32),
            pltpu.VMEM((ROWS_PER_TEC,), _f32),
            pltpu.VMEM_SHARED((NUM_TECS * 16,), _i32),
            pltpu.VMEM_SHARED((112,), _f32),
        ],
    )
    o_x1, o_y1, o_x2, o_y2, o_p = f(x1, y1, x2, y2, pp)
    out = jnp.stack([o_x1, o_y1, o_x2, o_y2, o_p], axis=1)
    return out[:N]


# speedup vs baseline: 35.6646x; 1.1014x over previous
import jax
import jax.numpy as jnp
from jax import lax
from jax.experimental import pallas as pl
from jax.experimental.pallas import tpu as pltpu
from jax.experimental.pallas import tpu_sc as plsc

N = 5000
N_PAD = 5120
NUM_TECS = 16
NB = N_PAD // 16
ROWS_PER_TEC = N_PAD // NUM_TECS
SLOTS = 328
SCORE_THRESHOLD = 0.05
NMS_THRESHOLD = 0.5

_f32 = jnp.float32
_i32 = jnp.int32


def _nms_body(x1h, y1h, x2h, y2h, ph,
              o_x1, o_y1, o_x2, o_y2, o_p,
              vx1, vy1, vx2, vy2, vp,
              sx1, sy1, sx2, sy2, sa,
              kept_all, tmp96, pall, tmp_ax,
              vo0, vo1, vo2, vo3, vo4,
              sh_part):
    c = lax.axis_index("c")
    w = lax.axis_index("s")

    @pl.when(c == 0)
    def _():
        pltpu.sync_copy(x1h, vx1.at[pl.ds(0, N_PAD)])
        pltpu.sync_copy(y1h, vy1.at[pl.ds(0, N_PAD)])
        pltpu.sync_copy(x2h, vx2.at[pl.ds(0, N_PAD)])
        pltpu.sync_copy(y2h, vy2.at[pl.ds(0, N_PAD)])
        pltpu.sync_copy(ph, vp)

        zf = jnp.zeros((16,), _f32)

        @pl.loop(0, SLOTS)
        def _zero(k):
            sx1[pl.ds(16 * k, 16)] = zf
            sy1[pl.ds(16 * k, 16)] = zf
            sx2[pl.ds(16 * k, 16)] = zf
            sy2[pl.ds(16 * k, 16)] = zf
            sa[pl.ds(16 * k, 16)] = zf

        lanes = lax.iota(_i32, 16)

        def block_body(b, l_len):
            base = 16 * b
            cx1 = vx1[pl.ds(base, 16)]
            cy1 = vy1[pl.ds(base, 16)]
            cx2 = vx2[pl.ds(base, 16)]
            cy2 = vy2[pl.ds(base, 16)]
            cp = vp[pl.ds(base, 16)]
            carea = jnp.maximum(cx2 - cx1, 0.0) * jnp.maximum(cy2 - cy1, 0.0)

            xw1 = jnp.full((16,), vx1[pl.ds(base + w, 16)][0], _f32)
            yw1 = jnp.full((16,), vy1[pl.ds(base + w, 16)][0], _f32)
            xw2 = jnp.full((16,), vx2[pl.ds(base + w, 16)][0], _f32)
            yw2 = jnp.full((16,), vy2[pl.ds(base + w, 16)][0], _f32)
            aw = (jnp.maximum(xw2 - xw1, 0.0)
                  * jnp.maximum(yw2 - yw1, 0.0))

            nk = jnp.maximum(l_len - w + 15, 0) // 16
            nk4 = (nk + 3) // 4

            def chunk_body(j, sup):
                for u in range(4):
                    r0 = 16 * (4 * j + u)
                    rx1 = sx1[pl.ds(r0, 16)]
                    ry1 = sy1[pl.ds(r0, 16)]
                    rx2 = sx2[pl.ds(r0, 16)]
                    ry2 = sy2[pl.ds(r0, 16)]
                    ra = sa[pl.ds(r0, 16)]
                    xx1 = jnp.maximum(cx1, rx1)
                    yy1 = jnp.maximum(cy1, ry1)
                    xx2 = jnp.minimum(cx2, rx2)
                    yy2 = jnp.minimum(cy2, ry2)
                    inter = (jnp.maximum(xx2 - xx1, 0.0)
                             * jnp.maximum(yy2 - yy1, 0.0))
                    union = carea + ra - inter
                    sup = sup + jnp.where(inter > NMS_THRESHOLD * union,
                                          1.0, 0.0)
                return sup

            sup = lax.fori_loop(0, nk4, chunk_body,
                                jnp.zeros((16,), _f32))

            tmp96[pl.ds(0, 16)] = sup
            tmp96[pl.ds(16, 16)] = xw1
            tmp96[pl.ds(32, 16)] = yw1
            tmp96[pl.ds(48, 16)] = xw2
            tmp96[pl.ds(64, 16)] = yw2
            tmp96[pl.ds(80, 16)] = aw
            pltpu.sync_copy(tmp96, sh_part.at[pl.ds(96 * w, 96)])
            plsc.subcore_barrier()

            pltpu.sync_copy(sh_part, pall)
            acc = pall[pl.ds(0, 16)]
            for i in range(1, NUM_TECS):
                acc = acc + pall[pl.ds(96 * i, 16)]
            kept = jnp.where((cp >= SCORE_THRESHOLD) & (acc == 0.0), 1, 0)
            pos = kept[0] * 0
            ax1 = jnp.zeros((16,), _f32)
            ay1 = jnp.zeros((16,), _f32)
            ax2 = jnp.zeros((16,), _f32)
            ay2 = jnp.zeros((16,), _f32)
            aa = jnp.zeros((16,), _f32)
            for t in range(16):
                xt1 = pall[pl.ds(96 * t + 16, 16)]
                yt1 = pall[pl.ds(96 * t + 32, 16)]
                xt2 = pall[pl.ds(96 * t + 48, 16)]
                yt2 = pall[pl.ds(96 * t + 64, 16)]
                at = pall[pl.ds(96 * t + 80, 16)]
                xx1 = jnp.maximum(cx1, xt1)
                yy1 = jnp.maximum(cy1, yt1)
                xx2 = jnp.minimum(cx2, xt2)
                yy2 = jnp.minimum(cy2, yt2)
                inter = (jnp.maximum(xx2 - xx1, 0.0)
                         * jnp.maximum(yy2 - yy1, 0.0))
                union = carea + at - inter
                rowt = (jnp.where(inter > NMS_THRESHOLD * union, 1, 0)
                        * jnp.where(lanes > t, 1, 0))
                kt = kept[t]
                ktv = jnp.full((16,), kt, _i32)
                kept = kept * (1 - rowt * ktv)
                ktf = kt.astype(_f32)
                hit = lanes == jnp.full((16,), pos, _i32)
                mf = jnp.where(hit, ktf, 0.0)
                ax1 = ax1 + mf * (xt1 - ax1)
                ay1 = ay1 + mf * (yt1 - ay1)
                ax2 = ax2 + mf * (xt2 - ax2)
                ay2 = ay2 + mf * (yt2 - ay2)
                aa = aa + mf * (at - aa)
                pos = pos + kt
            kept_all[pl.ds(base, 16)] = kept

            tmp_ax[pl.ds(0, 16)] = ax1
            tmp_ax[pl.ds(32, 16)] = ay1
            tmp_ax[pl.ds(64, 16)] = ax2
            tmp_ax[pl.ds(96, 16)] = ay2
            tmp_ax[pl.ds(128, 16)] = aa
            j_w = (w - (l_len & 15)) & 15

            @pl.when(j_w < pos)
            def _():
                slot = (l_len + j_w - w) // 16
                s0 = 16 * slot
                sx1[pl.ds(s0, 16)] = jnp.full(
                    (16,), tmp_ax[pl.ds(j_w, 16)][0], _f32)
                sy1[pl.ds(s0, 16)] = jnp.full(
                    (16,), tmp_ax[pl.ds(32 + j_w, 16)][0], _f32)
                sx2[pl.ds(s0, 16)] = jnp.full(
                    (16,), tmp_ax[pl.ds(64 + j_w, 16)][0], _f32)
                sy2[pl.ds(s0, 16)] = jnp.full(
                    (16,), tmp_ax[pl.ds(96 + j_w, 16)][0], _f32)
                sa[pl.ds(s0, 16)] = jnp.full(
                    (16,), tmp_ax[pl.ds(128 + j_w, 16)][0], _f32)

            return l_len + pos

        lax.fori_loop(0, NB, block_body, jnp.int32(0))

        row0 = ROWS_PER_TEC * w
        for g in range(ROWS_PER_TEC // 16):
            idx = row0 + 16 * g
            keepf = jnp.where(kept_all[pl.ds(idx, 16)] != 0, 1.0, 0.0)
            vo0[pl.ds(16 * g, 16)] = vx1[pl.ds(idx, 16)] * keepf
            vo1[pl.ds(16 * g, 16)] = vy1[pl.ds(idx, 16)] * keepf
            vo2[pl.ds(16 * g, 16)] = vx2[pl.ds(idx, 16)] * keepf
            vo3[pl.ds(16 * g, 16)] = vy2[pl.ds(idx, 16)] * keepf
            vo4[pl.ds(16 * g, 16)] = vp[pl.ds(idx, 16)] * keepf
        pltpu.sync_copy(vo0, o_x1.at[pl.ds(row0, ROWS_PER_TEC)])
        pltpu.sync_copy(vo1, o_y1.at[pl.ds(row0, ROWS_PER_TEC)])
        pltpu.sync_copy(vo2, o_x2.at[pl.ds(row0, ROWS_PER_TEC)])
        pltpu.sync_copy(vo3, o_y2.at[pl.ds(row0, ROWS_PER_TEC)])
        pltpu.sync_copy(vo4, o_p.at[pl.ds(row0, ROWS_PER_TEC)])


@jax.jit
def kernel(boxes, scores):
    probs = jax.nn.sigmoid(scores)
    order = jnp.argsort(-probs)
    b = jnp.take(boxes, order, axis=0)
    p = jnp.take(probs, order, axis=0)

    pad = N_PAD - N
    x1 = jnp.pad(b[:, 0], (0, pad))
    y1 = jnp.pad(b[:, 1], (0, pad))
    x2 = jnp.pad(b[:, 2], (0, pad))
    y2 = jnp.pad(b[:, 3], (0, pad))
    pp = jnp.pad(p, (0, pad))

    mesh = plsc.VectorSubcoreMesh(core_axis_name="c", subcore_axis_name="s")
    f = pl.kernel(
        _nms_body,
        out_type=[jax.ShapeDtypeStruct((N_PAD,), _f32)] * 5,
        mesh=mesh,
        scratch_types=[
            pltpu.VMEM((N_PAD + 16,), _f32),
            pltpu.VMEM((N_PAD + 16,), _f32),
            pltpu.VMEM((N_PAD + 16,), _f32),
            pltpu.VMEM((N_PAD + 16,), _f32),
            pltpu.VMEM((N_PAD,), _f32),
            pltpu.VMEM((SLOTS * 16,), _f32),
            pltpu.VMEM((SLOTS * 16,), _f32),
            pltpu.VMEM((SLOTS * 16,), _f32),
            pltpu.VMEM((SLOTS * 16,), _f32),
            pltpu.VMEM((SLOTS * 16,), _f32),
            pltpu.VMEM((N_PAD,), _i32),
            pltpu.VMEM((96,), _f32),
            pltpu.VMEM((NUM_TECS * 96,), _f32),
            pltpu.VMEM((160,), _f32),
            pltpu.VMEM((ROWS_PER_TEC,), _f32),
            pltpu.VMEM((ROWS_PER_TEC,), _f32),
            pltpu.VMEM((ROWS_PER_TEC,), _f32),
            pltpu.VMEM((ROWS_PER_TEC,), _f32),
            pltpu.VMEM((ROWS_PER_TEC,), _f32),
            pltpu.VMEM_SHARED((NUM_TECS * 96,), _f32),
        ],
    )
    o_x1, o_y1, o_x2, o_y2, o_p = f(x1, y1, x2, y2, pp)
    out = jnp.stack([o_x1, o_y1, o_x2, o_y2, o_p], axis=1)
    return out[:N]


# speedup vs baseline: 37.7518x; 1.0585x over previous
import jax
import jax.numpy as jnp
from jax import lax
from jax.experimental import pallas as pl
from jax.experimental.pallas import tpu as pltpu
from jax.experimental.pallas import tpu_sc as plsc

N = 5000
N_PAD = 5120
NUM_TECS = 16
NB = N_PAD // 16
ROWS_PER_TEC = N_PAD // NUM_TECS
SLOTS = 328
SCORE_THRESHOLD = 0.05
NMS_THRESHOLD = 0.5

_f32 = jnp.float32
_i32 = jnp.int32


def _nms_body(x1h, y1h, x2h, y2h, ph,
              o_x1, o_y1, o_x2, o_y2, o_p,
              vx1, vy1, vx2, vy2, vp,
              sx1, sy1, sx2, sy2, sa,
              kept_all, tmp96, pall, tmp_ax,
              vo0, vo1, vo2, vo3, vo4,
              sh_part):
    c = lax.axis_index("c")
    w = lax.axis_index("s")

    @pl.when(c == 0)
    def _():
        pltpu.sync_copy(x1h, vx1.at[pl.ds(0, N_PAD)])
        pltpu.sync_copy(y1h, vy1.at[pl.ds(0, N_PAD)])
        pltpu.sync_copy(x2h, vx2.at[pl.ds(0, N_PAD)])
        pltpu.sync_copy(y2h, vy2.at[pl.ds(0, N_PAD)])
        pltpu.sync_copy(ph, vp)

        zf = jnp.zeros((16,), _f32)

        @pl.loop(0, SLOTS)
        def _zero(k):
            sx1[pl.ds(16 * k, 16)] = zf
            sy1[pl.ds(16 * k, 16)] = zf
            sx2[pl.ds(16 * k, 16)] = zf
            sy2[pl.ds(16 * k, 16)] = zf
            sa[pl.ds(16 * k, 16)] = zf

        lanes = lax.iota(_i32, 16)

        def block_body(b, l_len):
            base = 16 * b
            cx1 = vx1[pl.ds(base, 16)]
            cy1 = vy1[pl.ds(base, 16)]
            cx2 = vx2[pl.ds(base, 16)]
            cy2 = vy2[pl.ds(base, 16)]
            cp = vp[pl.ds(base, 16)]
            carea = jnp.maximum(cx2 - cx1, 0.0) * jnp.maximum(cy2 - cy1, 0.0)

            xw1 = jnp.full((16,), vx1[pl.ds(base + w, 16)][0], _f32)
            yw1 = jnp.full((16,), vy1[pl.ds(base + w, 16)][0], _f32)
            xw2 = jnp.full((16,), vx2[pl.ds(base + w, 16)][0], _f32)
            yw2 = jnp.full((16,), vy2[pl.ds(base + w, 16)][0], _f32)
            aw = (jnp.maximum(xw2 - xw1, 0.0)
                  * jnp.maximum(yw2 - yw1, 0.0))

            nk = jnp.maximum(l_len - w + 15, 0) // 16
            nk4 = (nk + 3) // 4

            def chunk_body(j, sup):
                for u in range(4):
                    r0 = 16 * (4 * j + u)
                    rx1 = sx1[pl.ds(r0, 16)]
                    ry1 = sy1[pl.ds(r0, 16)]
                    rx2 = sx2[pl.ds(r0, 16)]
                    ry2 = sy2[pl.ds(r0, 16)]
                    ra = sa[pl.ds(r0, 16)]
                    xx1 = jnp.maximum(cx1, rx1)
                    yy1 = jnp.maximum(cy1, ry1)
                    xx2 = jnp.minimum(cx2, rx2)
                    yy2 = jnp.minimum(cy2, ry2)
                    inter = (jnp.maximum(xx2 - xx1, 0.0)
                             * jnp.maximum(yy2 - yy1, 0.0))
                    union = carea + ra - inter
                    sup = sup + jnp.where(inter > NMS_THRESHOLD * union,
                                          1.0, 0.0)
                return sup

            sup = lax.fori_loop(0, nk4, chunk_body,
                                jnp.zeros((16,), _f32))

            xx1 = jnp.maximum(cx1, xw1)
            yy1 = jnp.maximum(cy1, yw1)
            xx2 = jnp.minimum(cx2, xw2)
            yy2 = jnp.minimum(cy2, yw2)
            inter = (jnp.maximum(xx2 - xx1, 0.0)
                     * jnp.maximum(yy2 - yy1, 0.0))
            union = carea + aw - inter
            wv = jnp.full((16,), w, _i32)
            rowm = (jnp.where(inter > NMS_THRESHOLD * union, 1.0, 0.0)
                    * jnp.where(lanes > wv, 1.0, 0.0))

            tmp96[pl.ds(0, 16)] = sup
            tmp96[pl.ds(16, 16)] = rowm
            pltpu.sync_copy(tmp96.at[pl.ds(0, 32)],
                            sh_part.at[pl.ds(32 * w, 32)])
            plsc.subcore_barrier()

            pltpu.sync_copy(sh_part, pall)
            acc = pall[pl.ds(0, 16)]
            for i in range(1, NUM_TECS):
                acc = acc + pall[pl.ds(32 * i, 16)]
            keptf = jnp.where((cp >= SCORE_THRESHOLD) & (acc == 0.0),
                              1.0, 0.0)
            pos = w * 0
            ax1 = jnp.zeros((16,), _f32)
            ay1 = jnp.zeros((16,), _f32)
            ax2 = jnp.zeros((16,), _f32)
            ay2 = jnp.zeros((16,), _f32)
            aa = jnp.zeros((16,), _f32)
            for t in range(16):
                rowt = pall[pl.ds(32 * t + 16, 16)]
                ktf = keptf[t]
                keptf = keptf * (1.0 - rowt * jnp.full((16,), ktf, _f32))
                xt1 = jnp.full((16,), vx1[pl.ds(base + t, 16)][0], _f32)
                yt1 = jnp.full((16,), vy1[pl.ds(base + t, 16)][0], _f32)
                xt2 = jnp.full((16,), vx2[pl.ds(base + t, 16)][0], _f32)
                yt2 = jnp.full((16,), vy2[pl.ds(base + t, 16)][0], _f32)
                at = (jnp.maximum(xt2 - xt1, 0.0)
                      * jnp.maximum(yt2 - yt1, 0.0))
                hit = lanes == jnp.full((16,), pos, _i32)
                mf = jnp.where(hit, ktf, 0.0)
                ax1 = ax1 + mf * (xt1 - ax1)
                ay1 = ay1 + mf * (yt1 - ay1)
                ax2 = ax2 + mf * (xt2 - ax2)
                ay2 = ay2 + mf * (yt2 - ay2)
                aa = aa + mf * (at - aa)
                pos = pos + ktf.astype(_i32)
            kept_all[pl.ds(base, 16)] = jnp.where(keptf != 0.0, 1, 0)

            tmp_ax[pl.ds(0, 16)] = ax1
            tmp_ax[pl.ds(32, 16)] = ay1
            tmp_ax[pl.ds(64, 16)] = ax2
            tmp_ax[pl.ds(96, 16)] = ay2
            tmp_ax[pl.ds(128, 16)] = aa
            j_w = (w - (l_len & 15)) & 15

            @pl.when(j_w < pos)
            def _():
                slot = (l_len + j_w - w) // 16
                s0 = 16 * slot
                sx1[pl.ds(s0, 16)] = jnp.full(
                    (16,), tmp_ax[pl.ds(j_w, 16)][0], _f32)
                sy1[pl.ds(s0, 16)] = jnp.full(
                    (16,), tmp_ax[pl.ds(32 + j_w, 16)][0], _f32)
                sx2[pl.ds(s0, 16)] = jnp.full(
                    (16,), tmp_ax[pl.ds(64 + j_w, 16)][0], _f32)
                sy2[pl.ds(s0, 16)] = jnp.full(
                    (16,), tmp_ax[pl.ds(96 + j_w, 16)][0], _f32)
                sa[pl.ds(s0, 16)] = jnp.full(
                    (16,), tmp_ax[pl.ds(128 + j_w, 16)][0], _f32)

            return l_len + pos

        lax.fori_loop(0, NB, block_body, jnp.int32(0))

        row0 = ROWS_PER_TEC * w
        for g in range(ROWS_PER_TEC // 16):
            idx = row0 + 16 * g
            keepf = jnp.where(kept_all[pl.ds(idx, 16)] != 0, 1.0, 0.0)
            vo0[pl.ds(16 * g, 16)] = vx1[pl.ds(idx, 16)] * keepf
            vo1[pl.ds(16 * g, 16)] = vy1[pl.ds(idx, 16)] * keepf
            vo2[pl.ds(16 * g, 16)] = vx2[pl.ds(idx, 16)] * keepf
            vo3[pl.ds(16 * g, 16)] = vy2[pl.ds(idx, 16)] * keepf
            vo4[pl.ds(16 * g, 16)] = vp[pl.ds(idx, 16)] * keepf
        pltpu.sync_copy(vo0, o_x1.at[pl.ds(row0, ROWS_PER_TEC)])
        pltpu.sync_copy(vo1, o_y1.at[pl.ds(row0, ROWS_PER_TEC)])
        pltpu.sync_copy(vo2, o_x2.at[pl.ds(row0, ROWS_PER_TEC)])
        pltpu.sync_copy(vo3, o_y2.at[pl.ds(row0, ROWS_PER_TEC)])
        pltpu.sync_copy(vo4, o_p.at[pl.ds(row0, ROWS_PER_TEC)])


@jax.jit
def kernel(boxes, scores):
    probs = jax.nn.sigmoid(scores)
    order = jnp.argsort(-probs)
    b = jnp.take(boxes, order, axis=0)
    p = jnp.take(probs, order, axis=0)

    pad = N_PAD - N
    x1 = jnp.pad(b[:, 0], (0, pad))
    y1 = jnp.pad(b[:, 1], (0, pad))
    x2 = jnp.pad(b[:, 2], (0, pad))
    y2 = jnp.pad(b[:, 3], (0, pad))
    pp = jnp.pad(p, (0, pad))

    mesh = plsc.VectorSubcoreMesh(core_axis_name="c", subcore_axis_name="s")
    f = pl.kernel(
        _nms_body,
        out_type=[jax.ShapeDtypeStruct((N_PAD,), _f32)] * 5,
        mesh=mesh,
        scratch_types=[
            pltpu.VMEM((N_PAD + 16,), _f32),
            pltpu.VMEM((N_PAD + 16,), _f32),
            pltpu.VMEM((N_PAD + 16,), _f32),
            pltpu.VMEM((N_PAD + 16,), _f32),
            pltpu.VMEM((N_PAD,), _f32),
            pltpu.VMEM((SLOTS * 16,), _f32),
            pltpu.VMEM((SLOTS * 16,), _f32),
            pltpu.VMEM((SLOTS * 16,), _f32),
            pltpu.VMEM((SLOTS * 16,), _f32),
            pltpu.VMEM((SLOTS * 16,), _f32),
            pltpu.VMEM((N_PAD,), _i32),
            pltpu.VMEM((96,), _f32),
            pltpu.VMEM((NUM_TECS * 32,), _f32),
            pltpu.VMEM((160,), _f32),
            pltpu.VMEM((ROWS_PER_TEC,), _f32),
            pltpu.VMEM((ROWS_PER_TEC,), _f32),
            pltpu.VMEM((ROWS_PER_TEC,), _f32),
            pltpu.VMEM((ROWS_PER_TEC,), _f32),
            pltpu.VMEM((ROWS_PER_TEC,), _f32),
            pltpu.VMEM_SHARED((NUM_TECS * 32,), _f32),
        ],
    )
    o_x1, o_y1, o_x2, o_y2, o_p = f(x1, y1, x2, y2, pp)
    out = jnp.stack([o_x1, o_y1, o_x2, o_y2, o_p], axis=1)
    return out[:N]
